# baseline (speedup 1.0000x reference)
import functools

import jax
import jax.numpy as jnp
from jax import lax
from jax.experimental import pallas as pl
from jax.experimental.pallas import tpu as pltpu
from jax.experimental.pallas import tpu_sc as plsc


def _argmin_body(zt_ref, ct_ref, idx_ref):
    zf = zt_ref[0]
    ct = ct_ref[...]
    k_codes = ct.shape[1]
    dn = (((1,), (0,)), ((), ()))

    s = jax.lax.dot_general(zf, ct, dn, preferred_element_type=jnp.float32)
    znorm = jnp.sum(zf * zf, axis=1, keepdims=True)
    cnorm = jnp.sum(ct * ct, axis=0, keepdims=True)
    d = (znorm - 2.0 * s) + cnorm

    minv = jnp.min(d, axis=1, keepdims=True)
    ii = jax.lax.broadcasted_iota(jnp.int32, d.shape, 1)
    idx = jnp.min(jnp.where(d == minv, ii, k_codes), axis=1, keepdims=True)
    idx_ref[0] = idx.reshape(1, idx.shape[0])


def _make_sc_gather(n_tokens, dim, chunk):
    info = plsc.get_sparse_core_info()
    nw = info.num_cores * info.num_subcores
    per_w = n_tokens // nw
    mesh = plsc.VectorSubcoreMesh(core_axis_name="c", subcore_axis_name="s")

    @functools.partial(
        pl.kernel, mesh=mesh,
        out_type=jax.ShapeDtypeStruct((n_tokens, dim), jnp.float32),
        scratch_types=[
            pltpu.VMEM((chunk,), jnp.int32),
            pltpu.VMEM((chunk, dim), jnp.float32),
            pltpu.SemaphoreType.DMA,
        ],
    )
    def gather(table_hbm, idx_hbm, out_hbm, idx_v, rows_v, sem):
        wid = lax.axis_index("s") * info.num_cores + lax.axis_index("c")
        base = wid * per_w
        for j in range(per_w // chunk):
            off = base + j * chunk
            pltpu.sync_copy(idx_hbm.at[pl.ds(off, chunk)], idx_v)
            pltpu.async_copy(table_hbm.at[idx_v], rows_v, sem).wait()
            pltpu.sync_copy(rows_v, out_hbm.at[pl.ds(off, chunk)])

    return gather


def kernel(z, codebook):
    b, c, h, w = z.shape
    hw = h * w
    k = codebook.shape[0]
    zt = z.transpose(0, 2, 3, 1).reshape(b, hw, c)
    ct = codebook.T

    idx3 = pl.pallas_call(
        _argmin_body,
        grid=(b,),
        in_specs=[
            pl.BlockSpec((1, hw, c), lambda i: (i, 0, 0)),
            pl.BlockSpec((c, k), lambda i: (0, 0)),
        ],
        out_specs=pl.BlockSpec((1, 1, hw), lambda i: (i, 0, 0)),
        out_shape=jax.ShapeDtypeStruct((b, 1, hw), jnp.int32),
    )(zt, ct)
    idx2 = idx3.reshape(b, hw)

    gather = _make_sc_gather(b * hw, c, 128)
    zq_flat = gather(codebook, idx2.reshape(b * hw))
    zq = zq_flat.reshape(b, h, w, c).transpose(0, 3, 1, 2)
    return zq, idx2


# speedup vs baseline: 1.5512x; 1.5512x over previous
import jax
import jax.numpy as jnp
from jax.experimental import pallas as pl


def _vq_body(zt_ref, ct_ref, cbbf_ref, zq_ref, idx_ref):
    zf = zt_ref[0]
    ct = ct_ref[...]
    k_codes = ct.shape[1]
    dn = (((1,), (0,)), ((), ()))

    s = jax.lax.dot_general(zf, ct, dn, preferred_element_type=jnp.float32)
    znorm = jnp.sum(zf * zf, axis=1, keepdims=True)
    cnorm = 0.25 * jnp.sum(ct * ct, axis=0, keepdims=True)
    d = (znorm + s) + cnorm

    minv = jnp.min(d, axis=1, keepdims=True)
    ii = jax.lax.broadcasted_iota(jnp.int32, d.shape, 1)
    idx = jnp.min(jnp.where(d == minv, ii, k_codes), axis=1, keepdims=True)

    oh = (ii == idx).astype(jnp.bfloat16)
    zq = jax.lax.dot_general(oh, cbbf_ref[...], dn,
                             preferred_element_type=jnp.float32)
    zq_ref[0] = zq
    idx_ref[0] = idx.reshape(1, idx.shape[0])


def kernel(z, codebook):
    b, c, h, w = z.shape
    hw = h * w
    k = codebook.shape[0]
    zt = z.transpose(0, 2, 3, 1).reshape(b, hw, c)
    ct = -2.0 * codebook.T
    cb_bf = codebook.astype(jnp.bfloat16)

    zq3, idx3 = pl.pallas_call(
        _vq_body,
        grid=(b,),
        in_specs=[
            pl.BlockSpec((1, hw, c), lambda i: (i, 0, 0)),
            pl.BlockSpec((c, k), lambda i: (0, 0)),
            pl.BlockSpec((k, c), lambda i: (0, 0)),
        ],
        out_specs=[
            pl.BlockSpec((1, hw, c), lambda i: (i, 0, 0)),
            pl.BlockSpec((1, 1, hw), lambda i: (i, 0, 0)),
        ],
        out_shape=[
            jax.ShapeDtypeStruct((b, hw, c), jnp.float32),
            jax.ShapeDtypeStruct((b, 1, hw), jnp.int32),
        ],
    )(zt, ct, cb_bf)
    zq = zq3.reshape(b, h, w, c).transpose(0, 3, 1, 2)
    return zq, idx3.reshape(b, hw)


# speedup vs baseline: 1.6266x; 1.0486x over previous
import jax
import jax.numpy as jnp
from jax.experimental import pallas as pl


def _vq_body(zt_ref, ct_ref, cbbf_ref, zq_ref, idx_ref):
    ct = ct_ref[...]
    k_codes = ct.shape[1]
    dn = (((1,), (0,)), ((), ()))
    cnorm = 0.25 * jnp.sum(ct * ct, axis=0, keepdims=True)

    for t in range(zt_ref.shape[0]):
        zf = zt_ref[t]
        s = jax.lax.dot_general(zf, ct, dn, preferred_element_type=jnp.float32)
        znorm = jnp.sum(zf * zf, axis=1, keepdims=True)
        d = (znorm + s) + cnorm

        minv = jnp.min(d, axis=1, keepdims=True)
        ii = jax.lax.broadcasted_iota(jnp.int32, d.shape, 1)
        idx = jnp.min(jnp.where(d == minv, ii, k_codes), axis=1, keepdims=True)

        oh = (ii == idx).astype(jnp.bfloat16)
        zq = jax.lax.dot_general(oh, cbbf_ref[...], dn,
                                 preferred_element_type=jnp.float32)
        zq_ref[t] = zq
        idx_ref[0, t] = idx.reshape(1, idx.shape[0])[0]


def kernel(z, codebook):
    b, c, h, w = z.shape
    hw = h * w
    k = codebook.shape[0]
    zt = z.transpose(0, 2, 3, 1).reshape(b, hw, c)
    ct = -2.0 * codebook.T
    cb_bf = codebook.astype(jnp.bfloat16)

    zq3, idx3 = pl.pallas_call(
        _vq_body,
        grid=(b // 2,),
        in_specs=[
            pl.BlockSpec((2, hw, c), lambda i: (i, 0, 0)),
            pl.BlockSpec((c, k), lambda i: (0, 0)),
            pl.BlockSpec((k, c), lambda i: (0, 0)),
        ],
        out_specs=[
            pl.BlockSpec((2, hw, c), lambda i: (i, 0, 0)),
            pl.BlockSpec((1, 2, hw), lambda i: (i, 0, 0)),
        ],
        out_shape=[
            jax.ShapeDtypeStruct((b, hw, c), jnp.float32),
            jax.ShapeDtypeStruct((b // 2, 2, hw), jnp.int32),
        ],
    )(zt, ct, cb_bf)
    zq = zq3.reshape(b, h, w, c).transpose(0, 3, 1, 2)
    return zq, idx3.reshape(b, hw)


# speedup vs baseline: 1.6283x; 1.0011x over previous
import jax
import jax.numpy as jnp
from jax.experimental import pallas as pl


def _vq_body(zt_ref, ct_ref, cbbf_ref, zq_ref, idx_ref):
    ct = ct_ref[...]
    k_codes = ct.shape[1]
    dn = (((1,), (0,)), ((), ()))
    cnorm = 0.25 * jnp.sum(ct * ct, axis=0, keepdims=True)

    for t in range(zt_ref.shape[0]):
        zf = zt_ref[t]
        s = jax.lax.dot_general(zf, ct, dn, preferred_element_type=jnp.float32)
        znorm = jnp.sum(zf * zf, axis=1, keepdims=True)
        d = (znorm + s) + cnorm

        minv = jnp.min(d, axis=1, keepdims=True)
        ii = jax.lax.broadcasted_iota(jnp.int32, d.shape, 1)
        idx = jnp.min(jnp.where(d == minv, ii, k_codes), axis=1, keepdims=True)

        oh = (ii == idx).astype(jnp.bfloat16)
        zq = jax.lax.dot_general(oh, cbbf_ref[...], dn,
                                 preferred_element_type=jnp.float32)
        zq_ref[t] = zq
        idx_ref[0, t] = idx.reshape(1, idx.shape[0])[0]


def kernel(z, codebook):
    b, c, h, w = z.shape
    hw = h * w
    k = codebook.shape[0]
    zt = z.transpose(0, 2, 3, 1).reshape(b, hw, c)
    ct = -2.0 * codebook.T
    cb_bf = codebook.astype(jnp.bfloat16)

    zq3, idx3 = pl.pallas_call(
        _vq_body,
        grid=(b // 4,),
        in_specs=[
            pl.BlockSpec((4, hw, c), lambda i: (i, 0, 0)),
            pl.BlockSpec((c, k), lambda i: (0, 0)),
            pl.BlockSpec((k, c), lambda i: (0, 0)),
        ],
        out_specs=[
            pl.BlockSpec((4, hw, c), lambda i: (i, 0, 0)),
            pl.BlockSpec((1, 4, hw), lambda i: (i, 0, 0)),
        ],
        out_shape=[
            jax.ShapeDtypeStruct((b, hw, c), jnp.float32),
            jax.ShapeDtypeStruct((b // 4, 4, hw), jnp.int32),
        ],
    )(zt, ct, cb_bf)
    zq = zq3.reshape(b, h, w, c).transpose(0, 3, 1, 2)
    return zq, idx3.reshape(b, hw)
